# Initial kernel scaffold; baseline (speedup 1.0000x reference)
import functools

import jax
import jax.numpy as jnp
from jax import lax
from jax.experimental import pallas as pl
from jax.experimental.pallas import tpu as pltpu
from jax.experimental.pallas import tpu_sc as plsc

N_USERS = 5000
N = 10000
D = 128
E = 320000
LAYERS = 3

NC = 2
NS = 16
DH = D // NC
N_PAD = 10240
RT = N_PAD // NS
K = 128
C = 158
E_PAD = NS * C * K

_mesh = plsc.VectorSubcoreMesh(
    core_axis_name="c", subcore_axis_name="s", num_cores=NC, num_subcores=NS
)


def _iota16():
    return lax.iota(jnp.int32, 16)


def _full16(v):
    return jnp.full((16,), v, dtype=jnp.int32)


def _vload(ref, r, q):
    return plsc.load_gather(ref, [_full16(r), _iota16() + 16 * q])


def _vstore(ref, r, q, val):
    plsc.store_scatter(ref, [_full16(r), _iota16() + 16 * q], val)


def _bcast(ref, r):
    return plsc.load_gather(ref, [_full16(r)])


def _nrsqrt(d):
    i = plsc.bitcast(d, jnp.int32)
    i = 0x5F3759DF - lax.shift_right_logical(i, 1)
    y = plsc.bitcast(i, jnp.float32)
    for _ in range(3):
        y = y * (1.5 - 0.5 * d * y * y)
    return y


def _body(x0f, src2f, dstf, zacc, zdeg, ones_h, out_f, y_f,
          acc, dego, degi, sv, dv, r0, r1, Sv, degb, av, bv, svv, onev,
          sem0, sem1):
    c = lax.axis_index("c")
    t = lax.axis_index("s")
    ob = c * N_PAD + t * RT
    bn = t * RT

    pltpu.sync_copy(src2f.at[c * NS + t], sv)
    pltpu.sync_copy(dstf.at[t], dv)
    pltpu.sync_copy(ones_h, onev)

    pltpu.sync_copy(zdeg, dego.at[pl.ds(t * 2 * RT, 2 * RT)])
    pltpu.sync_copy(zdeg.at[pl.ds(0, RT)], degi.at[pl.ds(bn, RT)])
    plsc.subcore_barrier()

    def _deg_body(j, carry):
        pltpu.sync_copy(onev, dego.at[sv.at[j]], add=True)
        pltpu.sync_copy(onev, degi.at[dv.at[j]], add=True)
        return carry

    lax.fori_loop(0, C, _deg_body, 0)
    plsc.subcore_barrier()

    pltpu.sync_copy(dego.at[pl.ds(ob, RT)], degb)

    def _a_body(g, carry):
        dvals = plsc.load_gather(degb, [16 * g + _iota16(), _iota16()])
        av[pl.ds(16 * g, 16)] = _nrsqrt(jnp.maximum(dvals, 1.0))
        return carry

    lax.fori_loop(0, RT // 16, _a_body, 0)
    pltpu.sync_copy(degi.at[pl.ds(bn, RT)], degb)

    def _b_body(g, carry):
        dvals = plsc.load_gather(degb, [16 * g + _iota16(), _iota16()])
        bvals = _nrsqrt(jnp.maximum(dvals, 1.0))
        bv[pl.ds(16 * g, 16)] = bvals
        svv[pl.ds(16 * g, 16)] = bvals * av[pl.ds(16 * g, 16)]
        return carry

    lax.fori_loop(0, RT // 16, _b_body, 0)

    pltpu.sync_copy(x0f.at[pl.ds(ob, RT)], Sv)
    for m in range(RT // K):
        def _y0_body(rr, carry, m=m):
            rg = m * K + rr
            aa = _bcast(av, rg)
            for q in range(DH // 16):
                _vstore(r1, rr, q, aa * _vload(Sv, rg, q))
            return carry

        lax.fori_loop(0, K, _y0_body, 0)
        pltpu.sync_copy(r1, y_f.at[pl.ds(ob + m * K, K)])

    for layer in range(LAYERS):
        last = layer == LAYERS - 1
        pltpu.sync_copy(zacc, acc.at[pl.ds(bn, RT)])
        plsc.subcore_barrier()

        def _edge_body(i, carry):
            j = 2 * i
            d0 = pltpu.async_copy(y_f.at[sv.at[j]], r0, sem0)
            d1 = pltpu.async_copy(y_f.at[sv.at[j + 1]], r1, sem1)
            d0.wait()
            pltpu.sync_copy(r0, acc.at[dv.at[j]], add=True)
            d1.wait()
            pltpu.sync_copy(r1, acc.at[dv.at[j + 1]], add=True)
            return carry

        lax.fori_loop(0, C // 2, _edge_body, 0)
        plsc.subcore_barrier()

        for m in range(RT // K):
            pltpu.sync_copy(acc.at[pl.ds(bn + m * K, K)], r0)

            def _ep_body(rr, carry, m=m, last=last):
                rg = m * K + rr
                bb = _bcast(bv, rg)
                ss = None if last else _bcast(svv, rg)
                for q in range(DH // 16):
                    aseg = _vload(r0, rr, q)
                    snew = _vload(Sv, rg, q) + bb * aseg
                    _vstore(Sv, rg, q, snew)
                    if last:
                        _vstore(r1, rr, q, snew * 0.25)
                    else:
                        _vstore(r1, rr, q, ss * aseg)
                return carry

            lax.fori_loop(0, K, _ep_body, 0)
            if last:
                pltpu.sync_copy(r1, out_f.at[pl.ds(ob + m * K, K)])
            else:
                pltpu.sync_copy(r1, y_f.at[pl.ds(ob + m * K, K)])


_sc_kernel = functools.partial(
    pl.kernel,
    out_type=(
        jax.ShapeDtypeStruct((NC * N_PAD, DH), jnp.float32),
        jax.ShapeDtypeStruct((NC * N_PAD, DH), jnp.float32),
    ),
    mesh=_mesh,
    scratch_types=[
        pltpu.VMEM_SHARED((N_PAD, DH), jnp.float32),
        pltpu.VMEM_SHARED((NC * N_PAD, 16), jnp.float32),
        pltpu.VMEM_SHARED((N_PAD, 16), jnp.float32),
        pltpu.VMEM((C, K), jnp.int32),
        pltpu.VMEM((C, K), jnp.int32),
        pltpu.VMEM((K, DH), jnp.float32),
        pltpu.VMEM((K, DH), jnp.float32),
        pltpu.VMEM((RT, DH), jnp.float32),
        pltpu.VMEM((RT, 16), jnp.float32),
        pltpu.VMEM((RT,), jnp.float32),
        pltpu.VMEM((RT,), jnp.float32),
        pltpu.VMEM((RT,), jnp.float32),
        pltpu.VMEM((K, 16), jnp.float32),
        pltpu.SemaphoreType.DMA,
        pltpu.SemaphoreType.DMA,
    ],
)(_body)


def kernel(user_emb, item_emb, edge_index):
    src = edge_index[0]
    dst = edge_index[1]
    x0 = jnp.zeros((N_PAD, D), jnp.float32)
    x0 = x0.at[:N_USERS].set(user_emb).at[N_USERS:N].set(item_emb)
    x0f = jnp.concatenate([x0[:, :DH], x0[:, DH:]], axis=0)
    pad = jnp.full((E_PAD - E,), N, dtype=jnp.int32)
    sp = jnp.concatenate([src, pad]).reshape(NS, C, K)
    dp = jnp.concatenate([dst, pad]).reshape(NS, C, K)
    src2 = jnp.concatenate([sp, sp + N_PAD], axis=0)
    zacc = jnp.zeros((RT, DH), jnp.float32)
    zdeg = jnp.zeros((2 * RT, 16), jnp.float32)
    ones_h = jnp.ones((K, 16), jnp.float32)
    out_f, _ = _sc_kernel(x0f, src2, dp, zacc, zdeg, ones_h)
    final = jnp.concatenate([out_f[:N], out_f[N_PAD:N_PAD + N]], axis=1)
    return (final[:N_USERS], user_emb, final[N_USERS:], item_emb)


# speedup vs baseline: 9.4416x; 9.4416x over previous
import functools

import jax
import jax.numpy as jnp
from jax import lax
from jax.experimental import pallas as pl
from jax.experimental.pallas import tpu as pltpu
from jax.experimental.pallas import tpu_sc as plsc

N_USERS = 5000
N = 10000
D = 128
E = 320000
LAYERS = 3

NC = 2
NS = 16
DH = D // NC
N_PAD = 10240
RT = N_PAD // NS
HR = N_PAD // 128
K = 128
C = 158
E_PAD = NS * C * K

_mesh = plsc.VectorSubcoreMesh(
    core_axis_name="c", subcore_axis_name="s", num_cores=NC, num_subcores=NS
)


def _nrsqrt(d):
    i = plsc.bitcast(d, jnp.int32)
    i = 0x5F3759DF - lax.shift_right_logical(i, 1)
    y = plsc.bitcast(i, jnp.float32)
    for _ in range(3):
        y = y * (1.5 - 0.5 * d * y * y)
    return y


def _splat(ref, rg):
    v = ref[pl.ds(rg, 16)]
    return jnp.full((16,), v[0], dtype=jnp.float32)


def _body(x0f, src2f, dstf, z1, z2, out_f, y0_f, y1_f, y2_f,
          acc, histo, histi, sv, dv, r0, r1, r2, histL, degb,
          avv, bvv, svv, rowidx, sem0, sem1):
    c = lax.axis_index("c")
    t = lax.axis_index("s")
    ob = c * N_PAD + t * RT
    bn = t * RT
    off = c * N_PAD

    pltpu.sync_copy(src2f.at[c * NS + t], sv)
    pltpu.sync_copy(dstf.at[t], dv)

    pltpu.sync_copy(z2, histo.at[pl.ds(t * (HR // NS), HR // NS)])
    pltpu.sync_copy(z2, histi.at[pl.ds(t * (HR // NS), HR // NS)])
    for h in range(8):
        rowidx[0, pl.ds(h * 16, 16)] = lax.iota(jnp.int32, 16) + h * 16

    ones16 = jnp.ones((16,), jnp.float32)

    def _zero_hist():
        def _z(g, carry):
            for h in range(8):
                histL[g, pl.ds(h * 16, 16)] = jnp.zeros((16,), jnp.float32)
            return carry

        lax.fori_loop(0, HR, _z, 0)

    def _accum_hist(ref, sub_off):
        def _h(j, carry):
            for i in range(K // 16):
                iv = ref[j, pl.ds(16 * i, 16)] - sub_off
                plsc.addupdate_scatter(
                    histL,
                    [lax.shift_right_logical(iv, 7), lax.bitwise_and(iv, 127)],
                    ones16,
                )
            return carry

        lax.fori_loop(0, C, _h, 0)

    plsc.subcore_barrier()
    _zero_hist()
    _accum_hist(sv, off)
    pltpu.sync_copy(histL, histo.at[rowidx.at[0, pl.ds(0, HR)]], add=True)
    _zero_hist()
    _accum_hist(dv, 0)
    pltpu.sync_copy(histL, histi.at[rowidx.at[0, pl.ds(0, HR)]], add=True)
    plsc.subcore_barrier()

    pltpu.sync_copy(histi.at[pl.ds(bn // 128, RT // 128)], degb)
    for g in range(RT // 16):
        dvals = degb[g // 8, pl.ds((g % 8) * 16, 16)]
        bvv[pl.ds(16 * g, 16)] = _nrsqrt(jnp.maximum(dvals, 1.0))
    pltpu.sync_copy(histo.at[pl.ds(bn // 128, RT // 128)], degb)
    for g in range(RT // 16):
        dvals = degb[g // 8, pl.ds((g % 8) * 16, 16)]
        avals = _nrsqrt(jnp.maximum(dvals, 1.0))
        avv[pl.ds(16 * g, 16)] = avals
        svv[pl.ds(16 * g, 16)] = avals * bvv[pl.ds(16 * g, 16)]

    for m in range(RT // K):
        pltpu.sync_copy(x0f.at[pl.ds(ob + m * K, K)], r0)

        def _y0_body(rr, carry, m=m):
            aa = _splat(avv, m * K + rr)
            for q in range(DH // 16):
                r1[rr, pl.ds(16 * q, 16)] = aa * r0[rr, pl.ds(16 * q, 16)]
            return carry

        lax.fori_loop(0, K, _y0_body, 0)
        pltpu.sync_copy(r1, y0_f.at[pl.ds(ob + m * K, K)])

    y_bufs = [y0_f, y1_f, y2_f]
    for layer in range(LAYERS):
        last = layer == LAYERS - 1
        y_in = y_bufs[layer]
        pltpu.sync_copy(z1, acc.at[pl.ds(bn, RT)])
        plsc.subcore_barrier()

        def _edge_body(i, carry, y_in=y_in):
            j = 2 * i
            d0 = pltpu.async_copy(y_in.at[sv.at[j]], r0, sem0)
            d1 = pltpu.async_copy(y_in.at[sv.at[j + 1]], r1, sem1)
            d0.wait()
            pltpu.sync_copy(r0, acc.at[dv.at[j]], add=True)
            d1.wait()
            pltpu.sync_copy(r1, acc.at[dv.at[j + 1]], add=True)
            return carry

        lax.fori_loop(0, C // 2, _edge_body, 0)
        plsc.subcore_barrier()

        if not last:
            y_out = y_bufs[layer + 1]
            for m in range(RT // K):
                pltpu.sync_copy(acc.at[pl.ds(bn + m * K, K)], r0)

                def _ep_body(rr, carry, m=m):
                    ss = _splat(svv, m * K + rr)
                    for q in range(DH // 16):
                        cs = pl.ds(16 * q, 16)
                        r0[rr, cs] = ss * r0[rr, cs]
                    return carry

                lax.fori_loop(0, K, _ep_body, 0)
                pltpu.sync_copy(r0, y_out.at[pl.ds(ob + m * K, K)])
        else:
            for m in range(RT // K):
                pltpu.sync_copy(y1_f.at[pl.ds(ob + m * K, K)], r1)
                pltpu.sync_copy(y2_f.at[pl.ds(ob + m * K, K)], r2)

                def _fa_body(rr, carry, m=m):
                    aa = _splat(avv, m * K + rr)
                    for q in range(DH // 16):
                        cs = pl.ds(16 * q, 16)
                        r1[rr, cs] = (r1[rr, cs] + r2[rr, cs]) / aa
                    return carry

                lax.fori_loop(0, K, _fa_body, 0)
                pltpu.sync_copy(acc.at[pl.ds(bn + m * K, K)], r0)
                pltpu.sync_copy(x0f.at[pl.ds(ob + m * K, K)], r2)

                def _fb_body(rr, carry, m=m):
                    bb = _splat(bvv, m * K + rr)
                    for q in range(DH // 16):
                        cs = pl.ds(16 * q, 16)
                        r0[rr, cs] = (
                            r2[rr, cs] + r1[rr, cs] + bb * r0[rr, cs]
                        ) * 0.25
                    return carry

                lax.fori_loop(0, K, _fb_body, 0)
                pltpu.sync_copy(r0, out_f.at[pl.ds(ob + m * K, K)])


_sc_kernel = functools.partial(
    pl.kernel,
    out_type=(
        jax.ShapeDtypeStruct((NC * N_PAD, DH), jnp.float32),
        jax.ShapeDtypeStruct((NC * N_PAD, DH), jnp.float32),
        jax.ShapeDtypeStruct((NC * N_PAD, DH), jnp.float32),
        jax.ShapeDtypeStruct((NC * N_PAD, DH), jnp.float32),
    ),
    mesh=_mesh,
    scratch_types=[
        pltpu.VMEM_SHARED((N_PAD, DH), jnp.float32),
        pltpu.VMEM_SHARED((HR, 128), jnp.float32),
        pltpu.VMEM_SHARED((HR, 128), jnp.float32),
        pltpu.VMEM((C, K), jnp.int32),
        pltpu.VMEM((C, K), jnp.int32),
        pltpu.VMEM((K, DH), jnp.float32),
        pltpu.VMEM((K, DH), jnp.float32),
        pltpu.VMEM((K, DH), jnp.float32),
        pltpu.VMEM((HR, 128), jnp.float32),
        pltpu.VMEM((RT // 128, 128), jnp.float32),
        pltpu.VMEM((RT + 16,), jnp.float32),
        pltpu.VMEM((RT + 16,), jnp.float32),
        pltpu.VMEM((RT + 16,), jnp.float32),
        pltpu.VMEM((1, 128), jnp.int32),
        pltpu.SemaphoreType.DMA,
        pltpu.SemaphoreType.DMA,
    ],
    compiler_params=pltpu.CompilerParams(
        use_tc_tiling_on_sc=False, needs_layout_passes=False
    ),
)(_body)


def kernel(user_emb, item_emb, edge_index):
    src = edge_index[0]
    dst = edge_index[1]
    x0 = jnp.zeros((N_PAD, D), jnp.float32)
    x0 = x0.at[:N_USERS].set(user_emb).at[N_USERS:N].set(item_emb)
    x0f = jnp.concatenate([x0[:, :DH], x0[:, DH:]], axis=0)
    pad = jnp.full((E_PAD - E,), N, dtype=jnp.int32)
    sp = jnp.concatenate([src, pad]).reshape(NS, C, K)
    dp = jnp.concatenate([dst, pad]).reshape(NS, C, K)
    src2 = jnp.concatenate([sp, sp + N_PAD], axis=0)
    z1 = jnp.zeros((RT, DH), jnp.float32)
    z2 = jnp.zeros((HR // NS, 128), jnp.float32)
    out_f, _, _, _ = _sc_kernel(x0f, src2, dp, z1, z2)
    final = jnp.concatenate([out_f[:N], out_f[N_PAD:N_PAD + N]], axis=1)
    return (final[:N_USERS], user_emb, final[N_USERS:], item_emb)
